# baseline (speedup 1.0000x reference)
import functools

import jax
import jax.numpy as jnp
from jax import lax
from jax.experimental import pallas as pl
from jax.experimental.pallas import tpu as pltpu
from jax.experimental.pallas import tpu_sc as plsc

_RESOLUTIONS = (16, 20, 25, 32, 40, 50, 64, 80, 101, 128, 161, 203, 256, 322, 406, 512)
_NENC = 16384
_E = 2
_B = 262144
_NLVL = 16
_P1 = -1640531535
_P2 = 805459861
_NC, _NS, _LANES = 2, 16, 16
_NW = _NC * _NS
_BP = _B // _NW
_CHUNKS = _BP // _LANES


def _tec_body(x_ref, tabs_ref, out_ref, xs, tab0, tab1, acc0, acc1,
              ts0, ts1, as0, as1):
    wid = lax.axis_index("s") * _NC + lax.axis_index("c")
    base = wid * _BP
    pltpu.sync_copy(x_ref.at[:, pl.ds(base, _BP)], xs)

    tabs = (tab0, tab1)
    accs = (acc0, acc1)
    tsem = (ts0, ts1)
    asem = (as0, as1)
    tab_cp = [pltpu.async_copy(tabs_ref.at[0], tab0, ts0), None]
    acc_cp = [None, None]

    for l, res in enumerate(_RESOLUTIONS):
        nenc = min(res ** 3, _NENC)
        dense = res ** 3 <= _NENC
        pb = l % 2
        tab = tabs[pb]
        acc = accs[pb]
        tab_cp[pb].wait()
        if l + 1 < _NLVL:
            tab_cp[1 - pb] = pltpu.async_copy(
                tabs_ref.at[l + 1], tabs[1 - pb], tsem[1 - pb])
        if acc_cp[pb] is not None:
            acc_cp[pb].wait()

        plsc.subcore_barrier()
        a = jnp.float32(res * 0.5)
        b = jnp.float32((res - 1) * 0.5)

        @plsc.parallel_loop(0, _BP, _LANES, unroll=1)
        def chunk(i, res=res, nenc=nenc, dense=dense, a=a, b=b,
                  tab=tab, acc=acc):
            s = pl.ds(i, _LANES)
            gx = xs[0, s] * a + b
            gy = xs[1, s] * a + b
            gz = xs[2, s] * a + b
            x0 = gx.astype(jnp.int32)
            y0 = gy.astype(jnp.int32)
            z0 = gz.astype(jnp.int32)
            wx = gx - x0.astype(jnp.float32)
            wy = gy - y0.astype(jnp.float32)
            wz = gz - z0.astype(jnp.float32)
            x1 = x0 + 1
            y1 = y0 + 1
            z1 = z0 + 1
            wx1 = jnp.where(x1 < res, wx, jnp.float32(0.0))
            wy1 = jnp.where(y1 < res, wy, jnp.float32(0.0))
            wz1 = jnp.where(z1 < res, wz, jnp.float32(0.0))
            x1c = jnp.minimum(x1, res - 1)
            y1c = jnp.minimum(y1, res - 1)
            z1c = jnp.minimum(z1, res - 1)
            wxs = (jnp.float32(1.0) - wx, wx1)
            wys = (jnp.float32(1.0) - wy, wy1)
            wzs = (jnp.float32(1.0) - wz, wz1)
            if dense:
                zc = (z0 * (res * res), z1c * (res * res))
                yc = (y0 * res, y1c * res)
                xc = (x0, x1c)
            else:
                m = nenc - 1
                zc = (z0, z1c)
                yc = (y0 * _P1, y1c * _P1)
                xc = ((x0 * _P2) & m, (x1c * _P2) & m)
            wxp = (plsc.pack(wxs[0], wxs[0], format=plsc.PackFormat.INTERLEAVED),
                   plsc.pack(wxs[1], wxs[1], format=plsc.PackFormat.INTERLEAVED))
            apk = jnp.zeros((2 * _LANES,), jnp.bfloat16)
            for dz in (0, 1):
                for dy in (0, 1):
                    wzy = wzs[dz] * wys[dy]
                    wzyp = plsc.pack(wzy, wzy, format=plsc.PackFormat.INTERLEAVED)
                    zy = zc[dz] + yc[dy] if dense else (zc[dz] ^ yc[dy]) & m
                    for dx in (0, 1):
                        idx = zy + xc[dx] if dense else zy ^ xc[dx]
                        p = plsc.load_gather(tab, [idx])
                        epk = plsc.bitcast(p, jnp.bfloat16)
                        apk = apk + epk * (wzyp * wxp[dx])
            a0, a1 = plsc.unpack(apk, format=plsc.PackFormat.INTERLEAVED)
            acc[0, s] = a0
            acc[1, s] = a1

        acc_cp[pb] = pltpu.async_copy(
            acc, out_ref.at[l, :, pl.ds(base, _BP)], asem[pb])

    acc_cp[0].wait()
    acc_cp[1].wait()


def kernel(x, embs):
    def _pack(e):
        eb = jnp.pad(e, ((0, 0), (0, _NENC - e.shape[1]))).astype(jnp.bfloat16)
        bits = eb.view(jnp.uint16).astype(jnp.uint32)
        return (bits[0] | (bits[1] << 16)).astype(jnp.int32)

    tabs = jnp.stack([_pack(e) for e in embs])
    xt = x.T

    call = pl.kernel(
        _tec_body,
        out_type=jax.ShapeDtypeStruct((_NLVL, _E, _B), jnp.float32),
        mesh=plsc.VectorSubcoreMesh(
            core_axis_name="c", subcore_axis_name="s",
            num_cores=_NC, num_subcores=_NS),
        scratch_types=[
            pltpu.VMEM((3, _BP), jnp.float32),
            pltpu.VMEM((_NENC,), jnp.int32),
            pltpu.VMEM((_NENC,), jnp.int32),
            pltpu.VMEM((_E, _BP), jnp.float32),
            pltpu.VMEM((_E, _BP), jnp.float32),
            pltpu.SemaphoreType.DMA,
            pltpu.SemaphoreType.DMA,
            pltpu.SemaphoreType.DMA,
            pltpu.SemaphoreType.DMA,
        ],
        compiler_params=pltpu.CompilerParams(needs_layout_passes=False),
    )
    out_t = call(xt, tabs)
    return jnp.transpose(out_t, (2, 0, 1))


# speedup vs baseline: 1.0701x; 1.0701x over previous
import functools

import jax
import jax.numpy as jnp
from jax import lax
from jax.experimental import pallas as pl
from jax.experimental.pallas import tpu as pltpu
from jax.experimental.pallas import tpu_sc as plsc

_RESOLUTIONS = (16, 20, 25, 32, 40, 50, 64, 80, 101, 128, 161, 203, 256, 322, 406, 512)
_NENC = 16384
_E = 2
_B = 262144
_NLVL = 16
_P1 = -1640531535
_P2 = 805459861
_NC, _NS, _LANES = 2, 16, 16
_NW = _NC * _NS
_BP = _B // _NW
_CHUNKS = _BP // _LANES


def _tec_body(x_ref, tabs_ref, out_ref, xs, tab0, tab1, acc0, acc1,
              ts0, ts1, as0, as1):
    wid = lax.axis_index("s") * _NC + lax.axis_index("c")
    base = wid * _BP
    pltpu.sync_copy(x_ref.at[:, pl.ds(base, _BP)], xs)

    tabs = (tab0, tab1)
    accs = (acc0, acc1)
    tsem = (ts0, ts1)
    asem = (as0, as1)
    tab_cp = [pltpu.async_copy(tabs_ref.at[0], tab0, ts0), None]
    acc_cp = [None, None]

    for l, res in enumerate(_RESOLUTIONS):
        nenc = min(res ** 3, _NENC)
        dense = res ** 3 <= _NENC
        pb = l % 2
        tab = tabs[pb]
        acc = accs[pb]
        tab_cp[pb].wait()
        if l + 1 < _NLVL:
            tab_cp[1 - pb] = pltpu.async_copy(
                tabs_ref.at[l + 1], tabs[1 - pb], tsem[1 - pb])
        if acc_cp[pb] is not None:
            acc_cp[pb].wait()

        a = jnp.float32(res * 0.5)
        b = jnp.float32((res - 1) * 0.5)

        @plsc.parallel_loop(0, _BP, _LANES, unroll=1)
        def chunk(i, res=res, nenc=nenc, dense=dense, a=a, b=b,
                  tab=tab, acc=acc):
            s = pl.ds(i, _LANES)
            gx = xs[0, s] * a + b
            gy = xs[1, s] * a + b
            gz = xs[2, s] * a + b
            x0 = gx.astype(jnp.int32)
            y0 = gy.astype(jnp.int32)
            z0 = gz.astype(jnp.int32)
            wx = gx - x0.astype(jnp.float32)
            wy = gy - y0.astype(jnp.float32)
            wz = gz - z0.astype(jnp.float32)
            x1 = x0 + 1
            y1 = y0 + 1
            z1 = z0 + 1
            mx = x1 < res
            my = y1 < res
            mz = z1 < res
            wx1 = jnp.where(mx, wx, jnp.float32(0.0))
            wy1 = jnp.where(my, wy, jnp.float32(0.0))
            wz1 = jnp.where(mz, wz, jnp.float32(0.0))
            x1c = jnp.where(mx, x1, res - 1)
            y1c = jnp.where(my, y1, res - 1)
            z1c = jnp.where(mz, z1, res - 1)
            wxs = (jnp.float32(1.0) - wx, wx1)
            wys = (jnp.float32(1.0) - wy, wy1)
            wzs = (jnp.float32(1.0) - wz, wz1)
            if dense:
                zc = (z0 * (res * res), z1c * (res * res))
                yc = (y0 * res, y1c * res)
                xc = (x0, x1c)
            else:
                m = nenc - 1
                zc = (z0, z1c)
                yc = (y0 * _P1, y1c * _P1)
                xc = ((x0 * _P2) & m, (x1c * _P2) & m)
            _pk = lambda v: plsc.pack(v, v, format=plsc.PackFormat.INTERLEAVED)
            wxpk = (_pk(wxs[0]), _pk(wxs[1]))
            wypk = (_pk(wys[0]), _pk(wys[1]))
            wzpk = (_pk(wzs[0]), _pk(wzs[1]))
            az = []
            for dz in (0, 1):
                ay = []
                for dy in (0, 1):
                    zy = zc[dz] + yc[dy] if dense else (zc[dz] ^ yc[dy]) & m
                    ex = []
                    for dx in (0, 1):
                        idx = zy + xc[dx] if dense else zy ^ xc[dx]
                        p = plsc.load_gather(tab, [idx])
                        ex.append(plsc.bitcast(p, jnp.bfloat16))
                    ay.append(ex[0] * wxpk[0] + ex[1] * wxpk[1])
                az.append(ay[0] * wypk[0] + ay[1] * wypk[1])
            apk = az[0] * wzpk[0] + az[1] * wzpk[1]
            a0, a1 = plsc.unpack(apk, format=plsc.PackFormat.INTERLEAVED)
            acc[0, s] = a0
            acc[1, s] = a1

        acc_cp[pb] = pltpu.async_copy(
            acc, out_ref.at[l, :, pl.ds(base, _BP)], asem[pb])

    acc_cp[0].wait()
    acc_cp[1].wait()


def kernel(x, embs):
    def _pack(e):
        eb = jnp.pad(e, ((0, 0), (0, _NENC - e.shape[1]))).astype(jnp.bfloat16)
        bits = eb.view(jnp.uint16).astype(jnp.uint32)
        return (bits[0] | (bits[1] << 16)).astype(jnp.int32)

    tabs = jnp.stack([_pack(e) for e in embs])
    xt = x.T

    call = pl.kernel(
        _tec_body,
        out_type=jax.ShapeDtypeStruct((_NLVL, _E, _B), jnp.float32),
        mesh=plsc.VectorSubcoreMesh(
            core_axis_name="c", subcore_axis_name="s",
            num_cores=_NC, num_subcores=_NS),
        scratch_types=[
            pltpu.VMEM((3, _BP), jnp.float32),
            pltpu.VMEM((_NENC,), jnp.int32),
            pltpu.VMEM((_NENC,), jnp.int32),
            pltpu.VMEM((_E, _BP), jnp.float32),
            pltpu.VMEM((_E, _BP), jnp.float32),
            pltpu.SemaphoreType.DMA,
            pltpu.SemaphoreType.DMA,
            pltpu.SemaphoreType.DMA,
            pltpu.SemaphoreType.DMA,
        ],
        compiler_params=pltpu.CompilerParams(needs_layout_passes=False),
    )
    out_t = call(xt, tabs)
    return jnp.transpose(out_t, (2, 0, 1))
